# Initial kernel scaffold; baseline (speedup 1.0000x reference)
import jax
import jax.numpy as jnp
from jax.experimental import pallas as pl


def kernel(x, edge_index, W, b):
    raise NotImplementedError("write your pallas kernel here")


# speedup vs baseline: 41.8823x; 41.8823x over previous
import functools

import jax
import jax.numpy as jnp
from jax import lax
from jax.experimental import pallas as pl
from jax.experimental.pallas import tpu as pltpu
from jax.experimental.pallas import tpu_sc as plsc

NC = 2
NS = 16
NW = NC * NS
LANES = 16

D = 48
C_REAL = 40
B = 128

def _mesh():
    return plsc.VectorSubcoreMesh(
        core_axis_name="c", subcore_axis_name="s", num_cores=NC, num_subcores=NS
    )


def _make_deg_kernel(n_pad, e_pad):
    per_w = e_pad // NW
    chunks = per_w // B
    npw = n_pad // NS
    GRP = 8

    @functools.partial(
        pl.kernel,
        out_type=jax.ShapeDtypeStruct((NC, n_pad, LANES), jnp.float32),
        mesh=_mesh(),
        scratch_types=[
            pltpu.VMEM((chunks, B), jnp.int32),
            pltpu.VMEM((B, LANES), jnp.float32),
            pltpu.VMEM((npw, LANES), jnp.float32),
            pltpu.VMEM_SHARED((n_pad, LANES), jnp.float32),
            pltpu.SemaphoreType.DMA,
        ],
        compiler_params=pltpu.CompilerParams(use_tc_tiling_on_sc=False),
    )
    def deg_kernel(col_hbm, out_hbm, col_v, ones_v, zbuf, acc_sh, sem):
        c = lax.axis_index("c")
        s = lax.axis_index("s")
        wid = c * NS + s

        @pl.loop(0, B)
        def _(i):
            ones_v[i, :] = jnp.ones((LANES,), jnp.float32)

        @pl.loop(0, npw)
        def _(i):
            zbuf[i, :] = jnp.zeros((LANES,), jnp.float32)

        pltpu.sync_copy(zbuf, acc_sh.at[pl.ds(s * npw, npw)])
        pltpu.sync_copy(col_hbm.at[wid], col_v)
        plsc.subcore_barrier()

        @pl.loop(0, chunks // GRP)
        def _(k):
            descs = [
                pltpu.async_copy(
                    ones_v, acc_sh.at[col_v.at[GRP * k + j]], sem, add=True
                )
                for j in range(GRP)
            ]
            for d in descs:
                d.wait()

        plsc.subcore_barrier()
        pltpu.sync_copy(
            acc_sh.at[pl.ds(s * npw, npw)],
            out_hbm.at[c].at[pl.ds(s * npw, npw)],
        )

    return deg_kernel


def _make_hop_kernel(n_pad, e_pad):
    per_w = e_pad // NW
    chunks = per_w // B
    npw = n_pad // NS
    NBUF = 4

    @functools.partial(
        pl.kernel,
        out_type=jax.ShapeDtypeStruct((NC, n_pad, D), jnp.float32),
        mesh=_mesh(),
        scratch_types=[
            pltpu.VMEM((chunks, B), jnp.int32),
            pltpu.VMEM((chunks, B), jnp.int32),
            [pltpu.VMEM((B, D), jnp.float32)] * NBUF,
            pltpu.VMEM((B, D), jnp.float32),
            pltpu.VMEM_SHARED((n_pad, D), jnp.float32),
            pltpu.VMEM_SHARED((n_pad, D), jnp.float32),
            [pltpu.SemaphoreType.DMA] * NBUF,
            [pltpu.SemaphoreType.DMA] * NBUF,
        ],
        compiler_params=pltpu.CompilerParams(use_tc_tiling_on_sc=False),
    )
    def hop_kernel(
        z_hbm, row_hbm, col_hbm, out_hbm,
        row_v, col_v, msg, zbuf, z_sh, acc_sh, gs, ss,
    ):
        c = lax.axis_index("c")
        s = lax.axis_index("s")
        wid = c * NS + s

        pltpu.sync_copy(z_hbm.at[pl.ds(s * npw, npw)], z_sh.at[pl.ds(s * npw, npw)])

        @pl.loop(0, B)
        def _(i):
            for j in range(D // LANES):
                zbuf.at[i][pl.ds(j * LANES, LANES)] = jnp.zeros(
                    (LANES,), jnp.float32
                )

        @pl.loop(0, npw // B)
        def _(i):
            pltpu.sync_copy(zbuf, acc_sh.at[pl.ds(s * npw + i * B, B)])

        pltpu.sync_copy(row_hbm.at[wid], row_v)
        pltpu.sync_copy(col_hbm.at[wid], col_v)
        plsc.subcore_barrier()

        for b in range(NBUF - 1):
            pltpu.async_copy(z_sh.at[row_v.at[b]], msg[b], gs[b])
        pltpu.async_copy(
            zbuf, acc_sh.at[col_v.at[0]], ss[NBUF - 1], add=True
        )

        @pl.loop(0, chunks // NBUF)
        def _(k):
            for b in range(NBUF):
                i = NBUF * k + b
                b3 = (b + NBUF - 1) % NBUF
                pltpu.make_async_copy(z_sh.at[row_v.at[0]], msg[b], gs[b]).wait()
                pltpu.async_copy(msg[b], acc_sh.at[col_v.at[i]], ss[b], add=True)
                pltpu.make_async_copy(
                    zbuf, acc_sh.at[col_v.at[0]], ss[b3]
                ).wait()
                nxt = lax.rem(i + NBUF - 1, chunks)
                pltpu.async_copy(z_sh.at[row_v.at[nxt]], msg[b3], gs[b3])

        for b in range(NBUF - 1):
            pltpu.make_async_copy(z_sh.at[row_v.at[0]], msg[b], gs[b]).wait()
        pltpu.make_async_copy(
            zbuf, acc_sh.at[col_v.at[0]], ss[(chunks - 1) % NBUF]
        ).wait()

        plsc.subcore_barrier()
        pltpu.sync_copy(
            acc_sh.at[pl.ds(s * npw, npw)],
            out_hbm.at[c].at[pl.ds(s * npw, npw)],
        )

    return hop_kernel


def _matmul_scale_body(x_ref, w_ref, dp_ref, o_ref):
    y = jnp.dot(x_ref[...], w_ref[...], preferred_element_type=jnp.float32)
    o_ref[...] = y * lax.rsqrt(_deg_of(dp_ref))


def _deg_of(dp_ref):
    return dp_ref[0, :, :1] + dp_ref[1, :, :1] + 1.0


def _scale2_body(p_ref, z_ref, dp_ref, o_ref):
    u = p_ref[0] + p_ref[1] + z_ref[...]
    o_ref[...] = u / _deg_of(dp_ref)


def _final_body(q_ref, z_ref, dp_ref, b_ref, o_ref):
    u = q_ref[0] + q_ref[1] + z_ref[...]
    pre = u * lax.rsqrt(_deg_of(dp_ref))
    t = jnp.maximum(pre + b_ref[...], 0.0)
    mask = lax.broadcasted_iota(jnp.int32, t.shape, 1) < C_REAL
    neg = jnp.float32(-3.0e38)
    m = jnp.max(jnp.where(mask, t, neg), axis=1, keepdims=True)
    ssum = jnp.sum(jnp.where(mask, jnp.exp(t - m), 0.0), axis=1, keepdims=True)
    o_ref[...] = (t - m - jnp.log(ssum))[:, :C_REAL]


def kernel(x, edge_index, W, b):
    n, f_in = x.shape
    c_out = W.shape[0]
    e = edge_index.shape[1]

    bm = 2048
    n_pad = ((n + bm - 1) // bm) * bm
    if n_pad % NS:
        n_pad = ((n_pad + NS - 1) // NS) * NS
    e_pad = ((e + 4 * NW * B - 1) // (4 * NW * B)) * (4 * NW * B)

    x_p = jnp.pad(x, ((0, n_pad - n), (0, 0)))
    w_t = jnp.pad(W, ((0, D - c_out), (0, 0))).T
    b_p = jnp.pad(b, (0, D - c_out)).reshape(1, D)
    row = jnp.pad(edge_index[0], (0, e_pad - e))
    col = jnp.pad(edge_index[1], (0, e_pad - e), constant_values=n)
    chunks = e_pad // (NW * B)
    row3 = row.reshape(NW, chunks, B)
    col3 = col.reshape(NW, chunks, B)

    grid = (n_pad // bm,)
    dp = _make_deg_kernel(n_pad, e_pad)(col3)

    dp_spec = pl.BlockSpec((NC, bm, LANES), lambda i: (0, i, 0))
    z_spec = pl.BlockSpec((bm, D), lambda i: (i, 0))
    pp_spec = pl.BlockSpec((NC, bm, D), lambda i: (0, i, 0))
    zshape = jax.ShapeDtypeStruct((n_pad, D), jnp.float32)

    z1 = pl.pallas_call(
        _matmul_scale_body,
        grid=grid,
        in_specs=[
            pl.BlockSpec((bm, f_in), lambda i: (i, 0)),
            pl.BlockSpec((f_in, D), lambda i: (0, 0)),
            dp_spec,
        ],
        out_specs=z_spec,
        out_shape=zshape,
    )(x_p, w_t, dp)

    hop = _make_hop_kernel(n_pad, e_pad)
    p = hop(z1, row3, col3)

    z2 = pl.pallas_call(
        _scale2_body,
        grid=grid,
        in_specs=[pp_spec, z_spec, dp_spec],
        out_specs=z_spec,
        out_shape=zshape,
    )(p, z1, dp)

    q = hop(z2, row3, col3)

    bn = 2048
    out = pl.pallas_call(
        _final_body,
        grid=(pl.cdiv(n, bn),),
        in_specs=[
            pl.BlockSpec((NC, bn, D), lambda i: (0, i, 0)),
            pl.BlockSpec((bn, D), lambda i: (i, 0)),
            pl.BlockSpec((NC, bn, LANES), lambda i: (0, i, 0)),
            pl.BlockSpec((1, D), lambda i: (0, 0)),
        ],
        out_specs=pl.BlockSpec((bn, c_out), lambda i: (i, 0)),
        out_shape=jax.ShapeDtypeStruct((n, c_out), jnp.float32),
    )(q, z2, dp, b_p)

    return out
